# Initial kernel scaffold; baseline (speedup 1.0000x reference)
import jax
import jax.numpy as jnp
from jax.experimental import pallas as pl


def kernel(x, edge_index1, edge_index2, W1, aS1, aD1, W2, aS2, aD2):
    raise NotImplementedError("write your pallas kernel here")


# speedup vs baseline: 132.8897x; 132.8897x over previous
import functools

import jax
import jax.numpy as jnp
from jax import lax
from jax.experimental import pallas as pl
from jax.experimental.pallas import tpu as pltpu
from jax.experimental.pallas import tpu_sc as plsc

NN = 100000
FF = 10
EE = 3200000

NC, NS, LL = 2, 16, 16
NW = NC * NS
HP = 16
ONES_COL = FF

NPAD = 100352
SLICE = NPAD // NS
BLK = NPAD // 16

CHUNK = 256
SUB = CHUNK // 128
NCHUNK = 392
EP = NW * NCHUNK * CHUNK
PER_ROWS = (EP // NW) // 128

BIG_NEG = -1e38


def _alpha_mask(a, i):
    gidx = lax.broadcasted_iota(jnp.int32, a.shape, 0) + i * BLK
    return jnp.where(gidx < NN, a, BIG_NEG)


def _node_common(xb, w_ref, asv_ref, adv_ref, i,
                 hpad_ref, asrc_ref, adst_ref, bmax_ref):
    h = jnp.dot(xb, w_ref[...], preferred_element_type=jnp.float32)
    a_s = _alpha_mask(jnp.dot(h, asv_ref[...],
                              preferred_element_type=jnp.float32), i)
    a_d = _alpha_mask(jnp.dot(h, adv_ref[...],
                              preferred_element_type=jnp.float32), i)
    ones = jnp.ones((BLK, 1), jnp.float32)
    zeros = jnp.zeros((BLK, HP - FF - 1), jnp.float32)
    hpad_ref[...] = jnp.concatenate([h, ones, zeros], axis=1)
    asrc_ref[...] = a_s
    adst_ref[...] = a_d
    rowi = lax.broadcasted_iota(jnp.int32, (2, 16), 0)
    v = jnp.where(rowi == 0, jnp.max(a_s), jnp.max(a_d))

    @pl.when(i == 0)
    def _():
        bmax_ref[...] = jnp.full((2, 16), BIG_NEG, jnp.float32)

    bmax_ref[...] = jnp.maximum(bmax_ref[...], v)


def _node1_body(x_ref, w_ref, asv_ref, adv_ref,
                hpad_ref, asrc_ref, adst_ref, bmax_ref):
    i = pl.program_id(0)
    _node_common(x_ref[...], w_ref, asv_ref, adv_ref, i,
                 hpad_ref, asrc_ref, adst_ref, bmax_ref)


def _combine(acc0, acc1):
    t = acc0 + acc1
    return t[:, :FF] / (t[:, FF:FF + 1] + 1e-16)


def _node2_body(a0_ref, a1_ref, w_ref, asv_ref, adv_ref,
                hpad_ref, asrc_ref, adst_ref, bmax_ref):
    i = pl.program_id(0)
    xb = _combine(a0_ref[...], a1_ref[...])
    _node_common(xb, w_ref, asv_ref, adv_ref, i,
                 hpad_ref, asrc_ref, adst_ref, bmax_ref)


def _final_body(a0_ref, a1_ref, out_ref):
    out_ref[...] = _combine(a0_ref[...], a1_ref[...])


def _node_specs():
    wfull = pl.BlockSpec((FF, FF), lambda i: (0, 0))
    afull = pl.BlockSpec((FF, 1), lambda i: (0, 0))
    outs = (
        pl.BlockSpec((BLK, HP), lambda i: (i, 0)),
        pl.BlockSpec((BLK, 1), lambda i: (i, 0)),
        pl.BlockSpec((BLK, 1), lambda i: (i, 0)),
        pl.BlockSpec((2, 16), lambda i: (0, 0)),
    )
    out_shapes = (
        jax.ShapeDtypeStruct((NPAD, HP), jnp.float32),
        jax.ShapeDtypeStruct((NPAD, 1), jnp.float32),
        jax.ShapeDtypeStruct((NPAD, 1), jnp.float32),
        jax.ShapeDtypeStruct((2, 16), jnp.float32),
    )
    return wfull, afull, outs, out_shapes


def _node1(xp, w, asv, adv):
    wfull, afull, outs, out_shapes = _node_specs()
    return pl.pallas_call(
        _node1_body,
        grid=(16,),
        in_specs=[pl.BlockSpec((BLK, FF), lambda i: (i, 0)),
                  wfull, afull, afull],
        out_specs=outs,
        out_shape=out_shapes,
    )(xp, w, asv, adv)


def _node2(acc0, acc1, w, asv, adv):
    wfull, afull, outs, out_shapes = _node_specs()
    blk16 = pl.BlockSpec((BLK, HP), lambda i: (i, 0))
    return pl.pallas_call(
        _node2_body,
        grid=(16,),
        in_specs=[blk16, blk16, wfull, afull, afull],
        out_specs=outs,
        out_shape=out_shapes,
    )(acc0, acc1, w, asv, adv)


def _final(acc0, acc1):
    blk16 = pl.BlockSpec((BLK, HP), lambda i: (i, 0))
    return pl.pallas_call(
        _final_body,
        grid=(16,),
        in_specs=[blk16, blk16],
        out_specs=pl.BlockSpec((BLK, FF), lambda i: (i, 0)),
        out_shape=jax.ShapeDtypeStruct((NPAD, FF), jnp.float32),
    )(acc0, acc1)


def _vlane(v, i):
    idx = jnp.full((LL, 1), i, jnp.int32)
    dn = lax.GatherDimensionNumbers(
        offset_dims=(), collapsed_slice_dims=(0,), start_index_map=(0,))
    return lax.gather(v, idx, dn, (1,),
                      mode=lax.GatherScatterMode.PROMISE_IN_BOUNDS)


def _edge_body(hpad_hbm, asrc_hbm, adst_hbm, bmax_hbm, srci_hbm, dsti_hbm,
               zeros_hbm, out_hbm,
               asrc_sh, adst_sh, acc_sh,
               srcv, dstv, aval, dval, hrows,
               srcv_b, dstv_b, aval_b, dval_b, hrows_b, bm_vm,
               sem_g, sem_h, sem_s, sem_gb, sem_hb, sem_sb):
    c = lax.axis_index("c")
    s = lax.axis_index("s")
    wid = s * NC + c
    r0 = s * SLICE
    pltpu.sync_copy(asrc_hbm.at[pl.ds(r0, SLICE)], asrc_sh.at[pl.ds(r0, SLICE)])
    pltpu.sync_copy(adst_hbm.at[pl.ds(r0, SLICE)], adst_sh.at[pl.ds(r0, SLICE)])
    pltpu.sync_copy(zeros_hbm.at[pl.ds(r0, SLICE)], acc_sh.at[pl.ds(r0, SLICE)])
    pltpu.sync_copy(bmax_hbm, bm_vm)
    plsc.subcore_barrier()

    ssum = bm_vm[0, :] + bm_vm[1, :]
    m_shift = jnp.maximum(ssum, jnp.float32(0.2) * ssum)

    base_row = wid * PER_ROWS

    def compute(avalx, dvalx, hrowsx):
        def scale(gg, carry2):
            sl = pl.ds(gg * LL, LL)
            e16 = avalx[sl] + dvalx[sl]
            e16 = jnp.maximum(e16, jnp.float32(0.2) * e16)
            ex16 = jnp.exp(e16 - m_shift)
            for i in range(LL):
                r = gg * LL + i
                hrowsx[r, :] = hrowsx[r, :] * _vlane(ex16, i)
            return carry2

        lax.fori_loop(0, CHUNK // LL, scale, 0)

    def issue(cidx, srcvx, dstvx, avalx, dvalx, hrowsx, sgx, shx):
        row = base_row + cidx * SUB
        pltpu.sync_copy(srci_hbm.at[pl.ds(row, SUB)], srcvx)
        pltpu.sync_copy(dsti_hbm.at[pl.ds(row, SUB)], dstvx)
        cps = []
        for j in range(SUB):
            cps.append(pltpu.async_copy(asrc_sh.at[srcvx.at[j]],
                                        avalx.at[pl.ds(j * 128, 128)], sgx))
            cps.append(pltpu.async_copy(adst_sh.at[dstvx.at[j]],
                                        dvalx.at[pl.ds(j * 128, 128)], sgx))
            cps.append(pltpu.async_copy(hpad_hbm.at[srcvx.at[j]],
                                        hrowsx.at[pl.ds(j * 128, 128)], shx))
        return cps

    def scatter(dstvx, hrowsx, ssx):
        return [pltpu.async_copy(hrowsx.at[pl.ds(j * 128, 128)],
                                 acc_sh.at[dstvx.at[j]], ssx,
                                 add=True) for j in range(SUB)]

    def pair_body(k, carry):
        ga = issue(2 * k, srcv, dstv, aval, dval, hrows, sem_g, sem_h)
        gb = issue(2 * k + 1, srcv_b, dstv_b, aval_b, dval_b, hrows_b,
                   sem_gb, sem_hb)
        for cp in ga:
            cp.wait()
        compute(aval, dval, hrows)
        sa = scatter(dstv, hrows, sem_s)
        for cp in gb:
            cp.wait()
        compute(aval_b, dval_b, hrows_b)
        sb = scatter(dstv_b, hrows_b, sem_sb)
        for cp in sa:
            cp.wait()
        for cp in sb:
            cp.wait()
        return carry

    lax.fori_loop(0, NCHUNK // 2, pair_body, 0)
    plsc.subcore_barrier()
    pltpu.sync_copy(acc_sh.at[pl.ds(r0, SLICE)],
                    out_hbm.at[c, pl.ds(r0, SLICE)])


def _edge(hpad, asrc, adst, bmax, srci, dsti, zeros16):
    mesh = plsc.VectorSubcoreMesh(core_axis_name="c", subcore_axis_name="s")
    f = pl.kernel(
        _edge_body,
        out_type=jax.ShapeDtypeStruct((NC, NPAD, HP), jnp.float32),
        mesh=mesh,
        compiler_params=pltpu.CompilerParams(use_tc_tiling_on_sc=False),
        scratch_types=[
            pltpu.VMEM_SHARED((NPAD,), jnp.float32),
            pltpu.VMEM_SHARED((NPAD,), jnp.float32),
            pltpu.VMEM_SHARED((NPAD, HP), jnp.float32),
            pltpu.VMEM((SUB, 128), jnp.int32),
            pltpu.VMEM((SUB, 128), jnp.int32),
            pltpu.VMEM((CHUNK,), jnp.float32),
            pltpu.VMEM((CHUNK,), jnp.float32),
            pltpu.VMEM((CHUNK, HP), jnp.float32),
            pltpu.VMEM((SUB, 128), jnp.int32),
            pltpu.VMEM((SUB, 128), jnp.int32),
            pltpu.VMEM((CHUNK,), jnp.float32),
            pltpu.VMEM((CHUNK,), jnp.float32),
            pltpu.VMEM((CHUNK, HP), jnp.float32),
            pltpu.VMEM((2, 16), jnp.float32),
            pltpu.SemaphoreType.DMA,
            pltpu.SemaphoreType.DMA,
            pltpu.SemaphoreType.DMA,
            pltpu.SemaphoreType.DMA,
            pltpu.SemaphoreType.DMA,
            pltpu.SemaphoreType.DMA,
        ],
    )
    return f(hpad, asrc, adst, bmax, srci, dsti, zeros16)


def _pad_idx(idx):
    pad = jnp.full((EP - EE,), NN, jnp.int32)
    return jnp.concatenate([idx.astype(jnp.int32), pad]).reshape(EP // 128, 128)


def kernel(x, edge_index1, edge_index2, W1, aS1, aD1, W2, aS2, aD2):
    x = x.astype(jnp.float32)
    xp = jnp.zeros((NPAD, FF), jnp.float32).at[:NN].set(x)
    zeros16 = jnp.zeros((NPAD, HP), jnp.float32)
    s1 = _pad_idx(edge_index1[0])
    d1 = _pad_idx(edge_index1[1])
    s2 = _pad_idx(edge_index2[0])
    d2 = _pad_idx(edge_index2[1])

    hpad1, asrc1, adst1, bmax1 = _node1(
        xp, W1, aS1.reshape(FF, 1), aD1.reshape(FF, 1))
    acc1 = _edge(hpad1, asrc1.reshape(NPAD), adst1.reshape(NPAD), bmax1,
                 s1, d1, zeros16)
    hpad2, asrc2, adst2, bmax2 = _node2(
        acc1[0], acc1[1], W2, aS2.reshape(FF, 1), aD2.reshape(FF, 1))
    acc2 = _edge(hpad2, asrc2.reshape(NPAD), adst2.reshape(NPAD), bmax2,
                 s2, d2, zeros16)
    outp = _final(acc2[0], acc2[1])
    return outp[:NN]
